# baseline (speedup 1.0000x reference)
import jax
import jax.numpy as jnp
from jax.experimental import pallas as pl

B = 1024
H = 64
C = 10
EPS = 1e-5

TI_STATS = 8
TI_OUT = 8


def _log_softmax(y):
    m = jnp.max(y, axis=-1, keepdims=True)
    s = y - m
    return s - jnp.log(jnp.sum(jnp.exp(s), axis=-1, keepdims=True))


def _bn(x, g, b):
    m = jnp.mean(x, axis=0, keepdims=True)
    v = jnp.mean((x - m) * (x - m), axis=0, keepdims=True)
    return g * (x - m) * jax.lax.rsqrt(v + EPS) + b


def _dot_t(x, w):
    return jax.lax.dot_general(
        x, w, (((1,), (1,)), ((), ())), preferred_element_type=jnp.float32
    )


def _readout(x, g1, be1, w1, b1, g2, be2, w2, b2):
    x = _bn(x, g1, be1)
    x = jnp.maximum(_dot_t(x, w1) + b1, 0.0)
    x = _bn(x, g2, be2)
    return _log_softmax(_dot_t(x, w2) + b2)


def _prep_body(
    xo_ref, xc_ref,
    c_g1, c_b1, c_w1, c_bb1, c_g2, c_b2, c_w2, c_bb2,
    o_g1, o_b1, o_w1, o_bb1, o_g2, o_b2, o_w2, o_bb2,
    co_g1, co_b1, co_w1, co_bb1,
    xc_out, xo_out, u2_out, v_out,
):
    xo = xo_ref[...]
    xc = xc_ref[...]
    xc_out[...] = _readout(
        xc, c_g1[...], c_b1[...], c_w1[...], c_bb1[...],
        c_g2[...], c_b2[...], c_w2[...], c_bb2[...])
    xo_out[...] = _readout(
        xo, o_g1[...], o_b1[...], o_w1[...], o_bb1[...],
        o_g2[...], o_b2[...], o_w2[...], o_bb2[...])
    mo = jnp.mean(xo, axis=0, keepdims=True)
    mc = jnp.mean(xc, axis=0, keepdims=True)
    vo = jnp.mean((xo - mo) * (xo - mo), axis=0, keepdims=True)
    vc = jnp.mean((xc - mc) * (xc - mc), axis=0, keepdims=True)
    inv = jax.lax.rsqrt(vo + vc + EPS)
    g1 = co_g1[...]
    p = g1 * (xo - mo) * inv
    q = g1 * (xc - mc) * inv + co_b1[...]
    u = _dot_t(p, co_w1[...])
    v = _dot_t(q, co_w1[...]) + co_bb1[...]
    u2_out[...] = jnp.concatenate([u, u], axis=1)
    v_out[...] = v


def _stats_body(u2_ref, v2_ref, stats_ref):
    i = pl.program_id(0)
    r = jnp.maximum(u2_ref[...][:, None, :] + v2_ref[...][None, :, :], 0.0)
    s = jnp.sum(r, axis=(0, 1))
    ss = jnp.sum(r * r, axis=(0, 1))
    upd = jnp.concatenate(
        [s[None, :], ss[None, :], jnp.zeros((6, 2 * H), jnp.float32)], axis=0)

    @pl.when(i == 0)
    def _():
        stats_ref[...] = upd

    @pl.when(i > 0)
    def _():
        stats_ref[...] += upd


def _out_body(u2_ref, v2_ref, stats_ref, w2_ref, bb2_ref, g2_ref, be2_ref,
              out_ref):
    stats = stats_ref[...]
    n = float(B * B)
    m2 = (stats[0:1, 0:H] + stats[0:1, H:2 * H]) / n
    ex2 = (stats[1:2, 0:H] + stats[1:2, H:2 * H]) / n
    var = ex2 - m2 * m2
    scale = g2_ref[...] * jax.lax.rsqrt(var + EPS)
    w2 = w2_ref[...]
    w2e = w2 * scale
    beff = bb2_ref[...] + _dot_t(be2_ref[...] - m2 * scale, w2)
    z = jnp.zeros((C, H), jnp.float32)
    wblk = jnp.concatenate(
        [jnp.concatenate([w2e, z], axis=1),
         jnp.concatenate([z, w2e], axis=1)], axis=0)
    r = jnp.maximum(u2_ref[...][:, None, :] + v2_ref[...][None, :, :], 0.0)
    r = r.reshape(TI_OUT * (B // 2), 2 * H)
    y = _dot_t(r, wblk) + jnp.concatenate([beff, beff], axis=1)
    za = _log_softmax(y[:, 0:C])
    zb = _log_softmax(y[:, C:2 * C])
    out_ref[...] = jnp.concatenate([za, zb], axis=1)


def kernel(xo, xc, fc1_c_w, fc1_c_bias, fc2_c_w, fc2_c_bias,
           fc1_o_w, fc1_o_bias, fc2_o_w, fc2_o_bias,
           fc1_co_w, fc1_co_bias, fc2_co_w, fc2_co_bias,
           bn1_c_g, bn1_c_b, bn2_c_g, bn2_c_b,
           bn1_o_g, bn1_o_b, bn2_o_g, bn2_o_b,
           bn1_co_g, bn1_co_b, bn2_co_g, bn2_co_b):
    r2 = lambda a: a.reshape(1, -1)
    f32 = jnp.float32

    xc_logis, xo_logis, u2, v = pl.pallas_call(
        _prep_body,
        out_shape=[
            jax.ShapeDtypeStruct((B, C), f32),
            jax.ShapeDtypeStruct((B, C), f32),
            jax.ShapeDtypeStruct((B, 2 * H), f32),
            jax.ShapeDtypeStruct((B, H), f32),
        ],
    )(xo, xc,
      r2(bn1_c_g), r2(bn1_c_b), fc1_c_w, r2(fc1_c_bias),
      r2(bn2_c_g), r2(bn2_c_b), fc2_c_w, r2(fc2_c_bias),
      r2(bn1_o_g), r2(bn1_o_b), fc1_o_w, r2(fc1_o_bias),
      r2(bn2_o_g), r2(bn2_o_b), fc2_o_w, r2(fc2_o_bias),
      r2(bn1_co_g), r2(bn1_co_b), fc1_co_w, r2(fc1_co_bias))

    v2 = v.reshape(B // 2, 2 * H)

    stats = pl.pallas_call(
        _stats_body,
        grid=(B // TI_STATS,),
        in_specs=[
            pl.BlockSpec((TI_STATS, 2 * H), lambda i: (i, 0)),
            pl.BlockSpec((B // 2, 2 * H), lambda i: (0, 0)),
        ],
        out_specs=pl.BlockSpec((8, 2 * H), lambda i: (0, 0)),
        out_shape=jax.ShapeDtypeStruct((8, 2 * H), f32),
    )(u2, v2)

    xco2 = pl.pallas_call(
        _out_body,
        grid=(B // TI_OUT,),
        in_specs=[
            pl.BlockSpec((TI_OUT, 2 * H), lambda i: (i, 0)),
            pl.BlockSpec((B // 2, 2 * H), lambda i: (0, 0)),
            pl.BlockSpec((8, 2 * H), lambda i: (0, 0)),
            pl.BlockSpec((C, H), lambda i: (0, 0)),
            pl.BlockSpec((1, C), lambda i: (0, 0)),
            pl.BlockSpec((1, H), lambda i: (0, 0)),
            pl.BlockSpec((1, H), lambda i: (0, 0)),
        ],
        out_specs=pl.BlockSpec((TI_OUT * (B // 2), 2 * C), lambda i: (i, 0)),
        out_shape=jax.ShapeDtypeStruct((B * B // 2, 2 * C), f32),
    )(u2, v2, stats, fc2_co_w, r2(fc2_co_bias), r2(bn2_co_g), r2(bn2_co_b))

    xco_logis = xco2.reshape(B * B, C)
    return (xc_logis, xo_logis, xco_logis)
